# baseline (speedup 1.0000x reference)
import functools

import jax
import jax.numpy as jnp
from jax import lax
from jax.experimental import pallas as pl
from jax.experimental.pallas import tpu as pltpu, tpu_sc as plsc

NUM_ROWS = 100000
DIM = 64
BATCH = 4096
SEQ = 50
NC = 2
NS = 16
NW = NC * NS
BW = BATCH // NW
RB = DIM // 8
NBUF = 2
L = 16

_mesh = plsc.VectorSubcoreMesh(core_axis_name="c", subcore_axis_name="s")


@functools.partial(
    pl.kernel,
    out_type=jax.ShapeDtypeStruct((SEQ, RB, NW, 8 * BW), jnp.float32),
    mesh=_mesh,
    scratch_types=[
        pltpu.VMEM((SEQ, BW), jnp.int32),
        pltpu.VMEM((NBUF, BW, DIM), jnp.float32),
        pltpu.VMEM((NBUF, DIM * BW), jnp.float32),
        pltpu.SemaphoreType.DMA((NBUF,)),
        pltpu.SemaphoreType.DMA((NBUF,)),
    ],
    compiler_params=pltpu.CompilerParams(
        use_tc_tiling_on_sc=False, needs_layout_passes=False
    ),
)
def _gather_kernel(idx_hbm, table_hbm, out_hbm, idx_v, rows_v, stage_v, gsem, osem):
    wid = lax.axis_index("s") * NC + lax.axis_index("c")
    pltpu.sync_copy(idx_hbm.at[:, pl.ds(wid * BW, BW)], idx_v)

    iota = lax.iota(jnp.int32, L)
    fconst = [(kk * L + iota) * BW for kk in range(DIM // L)]

    def fire_gather(s, slot):
        pltpu.async_copy(table_hbm.at[idx_v.at[s]], rows_v.at[slot], gsem.at[slot])

    def fire_out(s, slot):
        for r in range(RB):
            pltpu.async_copy(
                stage_v.at[slot].at[pl.ds(r * 8 * BW, 8 * BW)],
                out_hbm.at[s, r, wid],
                osem.at[slot],
            )

    def wait_out(s, slot):
        for r in range(RB):
            pltpu.make_async_copy(
                stage_v.at[slot].at[pl.ds(r * 8 * BW, 8 * BW)],
                out_hbm.at[s, r, wid],
                osem.at[slot],
            ).wait()

    for slot in range(NBUF):
        fire_gather(slot, slot)

    @pl.loop(0, SEQ, step=NBUF)
    def _step(t):
        for slot in range(NBUF):
            s = t + slot
            pltpu.make_async_copy(
                table_hbm.at[idx_v.at[s]], rows_v.at[slot], gsem.at[slot]
            ).wait()

            @pl.when(s >= NBUF)
            def _wait_prev():
                wait_out(s - NBUF, slot)

            @pl.loop(0, BW, step=4)
            def _tr(b0):
                for bb in range(4):
                    b = b0 + bb
                    lane = jnp.full((L,), b, jnp.int32)
                    for kk in range(DIM // L):
                        v = rows_v.at[slot][b, pl.ds(kk * L, L)]
                        plsc.store_scatter(
                            stage_v.at[slot], [fconst[kk] + lane], v
                        )

            fire_out(s, slot)

            @pl.when(s + NBUF < SEQ)
            def _refill():
                fire_gather(s + NBUF, slot)

    for slot in range(NBUF):
        wait_out(SEQ - NBUF + slot, slot)


def kernel(input, weight):
    out_t = _gather_kernel(input.astype(jnp.int32).T, weight)
    out5 = out_t.reshape(SEQ, RB, NW, 8, BW)
    return out5.transpose(2, 4, 0, 1, 3).reshape(BATCH, SEQ, DIM)


# speedup vs baseline: 2.7660x; 2.7660x over previous
import functools

import jax
import jax.numpy as jnp
from jax import lax
from jax.experimental import pallas as pl
from jax.experimental.pallas import tpu as pltpu, tpu_sc as plsc

NUM_ROWS = 100000
DIM = 64
BATCH = 4096
SEQ = 50
NC = 2
NS = 16
NW = NC * NS
BW = BATCH // NW
RB = DIM // 8
NBUF = 2
L = 16

_mesh = plsc.VectorSubcoreMesh(core_axis_name="c", subcore_axis_name="s")


@functools.partial(
    pl.kernel,
    out_type=jax.ShapeDtypeStruct((SEQ, RB, NW, 8 * BW), jnp.float32),
    mesh=_mesh,
    scratch_types=[
        pltpu.VMEM((SEQ, BW), jnp.int32),
        pltpu.VMEM((NBUF, BW, DIM), jnp.float32),
        pltpu.VMEM((NBUF, DIM * BW), jnp.float32),
        pltpu.SemaphoreType.DMA((NBUF,)),
        pltpu.SemaphoreType.DMA((NBUF,)),
    ],
    compiler_params=pltpu.CompilerParams(
        use_tc_tiling_on_sc=False, needs_layout_passes=False
    ),
)
def _gather_kernel(idx_hbm, table_hbm, out_hbm, idx_v, rows_v, stage_v, gsem, osem):
    wid = lax.axis_index("s") * NC + lax.axis_index("c")
    pltpu.sync_copy(idx_hbm.at[:, pl.ds(wid * BW, BW)], idx_v)

    iota = lax.iota(jnp.int32, L)
    fconst = [(kk * L + iota) * BW for kk in range(DIM // L)]

    def fire_gather(s, slot):
        pltpu.async_copy(table_hbm.at[idx_v.at[s]], rows_v.at[slot], gsem.at[slot])

    def fire_out(s, slot):
        for r in range(RB):
            pltpu.async_copy(
                stage_v.at[slot].at[pl.ds(r * 8 * BW, 8 * BW)],
                out_hbm.at[s, r, wid],
                osem.at[slot],
            )

    def wait_out(s, slot):
        for r in range(RB):
            pltpu.make_async_copy(
                stage_v.at[slot].at[pl.ds(r * 8 * BW, 8 * BW)],
                out_hbm.at[s, r, wid],
                osem.at[slot],
            ).wait()

    for slot in range(NBUF):
        fire_gather(slot, slot)

    @pl.loop(0, SEQ, step=NBUF)
    def _step(t):
        for slot in range(NBUF):
            s = t + slot
            pltpu.make_async_copy(
                table_hbm.at[idx_v.at[s]], rows_v.at[slot], gsem.at[slot]
            ).wait()

            @pl.when(s >= NBUF)
            def _wait_prev():
                wait_out(s - NBUF, slot)

            @functools.partial(plsc.parallel_loop, 0, BW, unroll=4)
            def _tr(b):
                lane = jnp.full((L,), b, jnp.int32)
                for kk in range(DIM // L):
                    v = rows_v.at[slot][b, pl.ds(kk * L, L)]
                    plsc.store_scatter(
                        stage_v.at[slot], [fconst[kk] + lane], v
                    )

            fire_out(s, slot)

            @pl.when(s + NBUF < SEQ)
            def _refill():
                fire_gather(s + NBUF, slot)

    for slot in range(NBUF):
        wait_out(SEQ - NBUF + slot, slot)


def kernel(input, weight):
    out_t = _gather_kernel(input.astype(jnp.int32).T, weight)
    out5 = out_t.reshape(SEQ, RB, NW, 8, BW)
    return out5.transpose(2, 4, 0, 1, 3).reshape(BATCH, SEQ, DIM)
